# baseline (speedup 1.0000x reference)
import jax
import jax.numpy as jnp
from jax import lax
from jax.experimental import pallas as pl
from jax.experimental.pallas import tpu as pltpu
from jax.experimental.pallas import tpu_sc as plsc

M = 100000
N1 = 20000
D = 128
HISTORY_LEN = 30
TID_RANGE = 30000

NW = 32
CH = 3136
CT = 640
NSET = 512
NDROP = CH + CT + 16

SH = 112
NSEG_H = CH // SH
ST = 128
NSEG_T = CT // ST
NBUF = 4


def _sc_body(zr_hbm, t2_hbm, htids_hbm, hages_hbm, ttids_hbm,
             hfeat_hbm, tfeat_hbm, out_hbm,
             table_v, t2_v, tids_v, ages_v, ttids_v, drop_v, zrows_v,
             b0, b1, b2, b3, i0, i1, i2, i3, o0, o1, o2, o3, ssem,
             m1, m2, m3, m4, m5):
    wid = lax.axis_index("s") * 2 + lax.axis_index("c")
    bufs = [b0, b1, b2, b3]
    ins = [i0, i1, i2, i3]
    outs = [o0, o1, o2, o3]
    base_h = wid * CH
    start_h = jnp.minimum(base_h, M - CH)
    base_t = wid * CT
    start_t = jnp.minimum(base_t, N1 - CT)

    pltpu.async_copy(t2_hbm, t2_v, m1)
    pltpu.async_copy(htids_hbm.at[pl.ds(start_h, CH)], tids_v, m2)
    pltpu.async_copy(hages_hbm.at[pl.ds(start_h, CH)], ages_v, m3)
    pltpu.async_copy(ttids_hbm.at[pl.ds(start_t, CT)], ttids_v, m4)
    pltpu.async_copy(zr_hbm, zrows_v, m5)

    def h_src(g):
        start = jnp.minimum(base_h + g * SH, M - SH)
        return hfeat_hbm.at[pl.ds(start, SH), :]

    def h_dst(g):
        start = jnp.minimum(base_h + g * SH, M - SH)
        return out_hbm.at[pl.ds(start, SH), :]

    for b in range(NBUF):
        pltpu.async_copy(h_src(b), bufs[b].at[pl.ds(0, SH), :], ins[b])

    def _zt(i, _):
        table_v[pl.ds(i * 16, 16)] = jnp.zeros((16,), jnp.float32)
        return 0
    lax.fori_loop(0, TID_RANGE // 16, _zt, 0)

    pltpu.make_async_copy(t2_hbm, t2_v, m1).wait()

    def _scatter(j, _):
        idx = t2_v[pl.ds(j * 16, 16)]
        plsc.store_scatter(table_v, [idx], jnp.ones((16,), jnp.float32))
        return 0
    lax.fori_loop(0, NSET // 16, _scatter, 0)

    pltpu.make_async_copy(htids_hbm.at[pl.ds(start_h, CH)], tids_v, m2).wait()
    pltpu.make_async_copy(hages_hbm.at[pl.ds(start_h, CH)], ages_v, m3).wait()
    lanes = lax.iota(jnp.int32, 16)

    def _hist(k, cnt):
        sl = pl.ds(k * 16, 16)
        tid = tids_v[sl]
        hit = plsc.load_gather(table_v, [tid])
        age = ages_v[sl]
        keep = jnp.logical_and(
            jnp.logical_and(tid != 0, hit == 0.0), age <= HISTORY_LEN - 1)
        rows = start_h + k * 16 + lanes
        dropm = jnp.logical_and(jnp.logical_not(keep), rows >= base_h)
        plsc.store_compressed(drop_v.at[pl.ds(cnt, 16)], rows, mask=dropm)
        return cnt + jnp.sum(dropm.astype(jnp.int32))
    cnt_h = lax.fori_loop(0, CH // 16, _hist, jnp.int32(0))

    pltpu.make_async_copy(ttids_hbm.at[pl.ds(start_t, CT)], ttids_v, m4).wait()

    def _det(k, cnt):
        sl = pl.ds(k * 16, 16)
        tid = ttids_v[sl]
        hit = plsc.load_gather(table_v, [tid])
        keep = jnp.logical_and(tid != 0, hit == 0.0)
        rows = start_t + k * 16 + lanes
        dropm = jnp.logical_and(jnp.logical_not(keep), rows >= base_t)
        plsc.store_compressed(drop_v.at[pl.ds(cnt, 16)], rows + M, mask=dropm)
        return cnt + jnp.sum(dropm.astype(jnp.int32))
    cnt = lax.fori_loop(0, CT // 16, _det, cnt_h)

    @pl.loop(0, NSEG_H // NBUF)
    def _copy_h(o):
        for b in range(NBUF):
            g = o * NBUF + b
            pltpu.make_async_copy(h_src(0), bufs[b].at[pl.ds(0, SH), :],
                                  ins[b]).wait()
            pltpu.async_copy(bufs[b].at[pl.ds(0, SH), :], h_dst(g), outs[b])

            @pl.when(g + NBUF < NSEG_H)
            def _():
                pltpu.make_async_copy(bufs[b].at[pl.ds(0, SH), :], h_dst(0),
                                      outs[b]).wait()
                pltpu.async_copy(h_src(g + NBUF), bufs[b].at[pl.ds(0, SH), :],
                                 ins[b])

    for b in range(NBUF):
        pltpu.make_async_copy(bufs[b].at[pl.ds(0, SH), :], h_dst(0),
                              outs[b]).wait()

    pltpu.make_async_copy(zr_hbm, zrows_v, m5).wait()
    nfull_h = cnt_h // 16

    def _fire_h(c, _):
        idx = drop_v[pl.ds(c * 16, 16)]
        pltpu.async_copy(zrows_v, out_hbm.at[idx], ssem)
        return 0
    lax.fori_loop(0, nfull_h, _fire_h, 0)

    def t_src(g):
        start = jnp.minimum(base_t + g * ST, N1 - ST)
        return tfeat_hbm.at[pl.ds(start, ST), :]

    def t_dst(g):
        start = jnp.minimum(base_t + g * ST, N1 - ST)
        return out_hbm.at[pl.ds(M + start, ST), :]

    for b in range(NBUF):
        pltpu.async_copy(t_src(b), bufs[b], ins[b])
    for g in range(NSEG_T):
        b = g % NBUF
        pltpu.make_async_copy(t_src(0), bufs[b], ins[b]).wait()
        pltpu.async_copy(bufs[b], t_dst(g), outs[b])
        if g + NBUF < NSEG_T:
            pltpu.make_async_copy(bufs[b], t_dst(0), outs[b]).wait()
            pltpu.async_copy(t_src(g + NBUF), bufs[b], ins[b])
    for g in range(max(NSEG_T - NBUF, 0), NSEG_T):
        pltpu.make_async_copy(bufs[g % NBUF], t_dst(0), outs[g % NBUF]).wait()

    nfull = cnt // 16
    rem = cnt - nfull * 16

    def _fire(c, _):
        idx = drop_v[pl.ds(c * 16, 16)]
        pltpu.async_copy(zrows_v, out_hbm.at[idx], ssem)
        return 0
    lax.fori_loop(nfull_h, nfull, _fire, 0)

    @pl.when(rem > 0)
    def _():
        head = plsc.load_gather(drop_v, [jnp.full((16,), nfull * 16, jnp.int32)])
        tail = drop_v[pl.ds(nfull * 16, 16)]
        idx = jnp.where(lanes < rem, tail, head)
        pltpu.async_copy(zrows_v, out_hbm.at[idx], ssem)

    nchunks = nfull + jnp.where(rem > 0, 1, 0).astype(jnp.int32)

    def _drain(c, _):
        pltpu.make_async_copy(zrows_v, out_hbm.at[jnp.zeros((16,), jnp.int32)],
                              ssem).wait()
        return 0
    lax.fori_loop(0, nchunks, _drain, 0)


_sc_full = pl.kernel(
    _sc_body,
    out_type=jax.ShapeDtypeStruct((M + N1, D), jnp.float32),
    mesh=plsc.VectorSubcoreMesh(core_axis_name="c", subcore_axis_name="s"),
    compiler_params=pltpu.CompilerParams(needs_layout_passes=False),
    scratch_types=[
        pltpu.VMEM((TID_RANGE,), jnp.float32),
        pltpu.VMEM((NSET,), jnp.int32),
        pltpu.VMEM((CH,), jnp.int32),
        pltpu.VMEM((CH,), jnp.int32),
        pltpu.VMEM((CT,), jnp.int32),
        pltpu.VMEM((NDROP,), jnp.int32),
        pltpu.VMEM((16, D), jnp.float32),
    ]
    + [pltpu.VMEM((ST, D), jnp.float32)] * NBUF
    + [pltpu.SemaphoreType.DMA] * (2 * NBUF + 1 + 5),
)


def kernel(t1_feats, hist_feats, t1_tids, t2_tids, hist_tids, hist_ages):
    zrows = jnp.zeros((16, D), jnp.float32)
    return _sc_full(zrows, t2_tids[0], hist_tids[0], hist_ages,
                    t1_tids[0], hist_feats, t1_feats)
